# baseline (speedup 1.0000x reference)
import functools

import jax
import jax.numpy as jnp
from jax import lax
from jax.experimental import pallas as pl
from jax.experimental.pallas import tpu as pltpu
from jax.experimental.pallas import tpu_sc as plsc

NSUB = 16
NCORE = 2
EBLK = 128
F2 = 64
ROWT = 256

_HI = jax.lax.Precision.HIGHEST


def _make_aprop(nblk: int, npad: int):
    slab = npad + 16
    rows_per_sub = npad // NSUB
    nchunk = rows_per_sub // 128
    mesh = plsc.VectorSubcoreMesh(core_axis_name="c", subcore_axis_name="s")

    @functools.partial(
        pl.kernel,
        out_type=jax.ShapeDtypeStruct((NCORE, npad, F2), jnp.float32),
        mesh=mesh,
        scratch_types=[
            pltpu.VMEM((nblk + 4, EBLK), jnp.int32),
            pltpu.VMEM((nblk, EBLK), jnp.int32),
            pltpu.VMEM((4, EBLK, F2), jnp.float32),
            pltpu.VMEM_SHARED((slab, F2), jnp.float32),
            pltpu.SemaphoreType.DMA,
            pltpu.SemaphoreType.DMA,
            pltpu.SemaphoreType.DMA,
            pltpu.SemaphoreType.DMA,
        ],
        compiler_params=pltpu.CompilerParams(use_tc_tiling_on_sc=False),
    )
    def aprop(in_hbm, src_hbm, dst_hbm, zero_hbm, out_hbm,
              src_v, dst_v, gbuf, acc, *sems):
        c = lax.axis_index("c")
        s = lax.axis_index("s")
        pltpu.sync_copy(src_hbm.at[s], src_v)
        pltpu.sync_copy(dst_hbm.at[s], dst_v)
        pltpu.sync_copy(zero_hbm, gbuf.at[0])
        base = s * rows_per_sub
        for k in range(nchunk):
            pltpu.sync_copy(gbuf.at[0], acc.at[pl.ds(base + k * 128, 128)])

        @pl.when(s == NSUB - 1)
        def _():
            pltpu.sync_copy(gbuf.at[0].at[pl.ds(0, 16)], acc.at[pl.ds(npad, 16)])

        plsc.subcore_barrier()

        def run(in_h, out_h):
            for k in range(3):
                pltpu.async_copy(in_h.at[src_v.at[k]], gbuf.at[k], sems[k])

            def body(i, carry):
                j0 = i * 4
                for k in range(4):
                    j = j0 + k
                    g = gbuf.at[k]
                    pltpu.make_async_copy(
                        in_h.at[src_v.at[j]], g, sems[k]).wait()
                    pltpu.sync_copy(g, acc.at[dst_v.at[j]], add=True)
                    kn = (k + 3) % 4
                    pltpu.async_copy(
                        in_h.at[src_v.at[j + 3]], gbuf.at[kn], sems[kn])
                return carry

            lax.fori_loop(0, nblk // 4, body, 0)
            for k in range(3):
                pltpu.make_async_copy(
                    in_h.at[src_v.at[nblk + k]], gbuf.at[k], sems[k]).wait()
            plsc.subcore_barrier()
            for k in range(nchunk):
                r = base + k * 128
                pltpu.sync_copy(acc.at[pl.ds(r, 128)], out_h.at[pl.ds(r, 128)])

        @pl.when(c == 0)
        def _():
            run(in_hbm.at[0], out_hbm.at[0])

        @pl.when(c == 1)
        def _():
            run(in_hbm.at[1], out_hbm.at[1])

    return aprop


def _rowscale_split(a, svec, npad):
    F = a.shape[1]

    def body(a_ref, s_ref, o_ref):
        av = a_ref[...] * s_ref[...]
        o_ref[0] = av[:, :F2]
        o_ref[1] = av[:, F2:]

    return pl.pallas_call(
        body,
        grid=(npad // ROWT,),
        in_specs=[
            pl.BlockSpec((ROWT, F), lambda i: (i, 0)),
            pl.BlockSpec((ROWT, 1), lambda i: (i, 0)),
        ],
        out_specs=pl.BlockSpec((2, ROWT, F2), lambda i: (0, i, 0)),
        out_shape=jax.ShapeDtypeStruct((2, npad, F2), jnp.float32),
    )(a, svec)


def _rowscale_stacked(v, svec, npad):
    def body(v_ref, s_ref, o_ref):
        o_ref[...] = v_ref[...] * s_ref[...][None]

    return pl.pallas_call(
        body,
        grid=(npad // ROWT,),
        in_specs=[
            pl.BlockSpec((2, ROWT, F2), lambda i: (0, i, 0)),
            pl.BlockSpec((ROWT, 1), lambda i: (i, 0)),
        ],
        out_specs=pl.BlockSpec((2, ROWT, F2), lambda i: (0, i, 0)),
        out_shape=jax.ShapeDtypeStruct((2, npad, F2), jnp.float32),
    )(v, svec)


def _cheb_mix(t, v1_parts, v2_parts, svec, W, b, npad, emit_next):
    Fin = t.shape[1]
    H = W.shape[2]
    nparts = len(v1_parts)
    ngroups = H // (2 * F2)
    b2d = b.reshape(1, H)

    def body(*refs):
        t_ref = refs[0]
        v1_refs = refs[1:1 + nparts]
        v2_refs = refs[1 + nparts:1 + 2 * nparts]
        s_ref, w_ref, b_ref = refs[1 + 2 * nparts:4 + 2 * nparts]
        out_refs = refs[4 + 2 * nparts:]
        sv = s_ref[...]
        tt = t_ref[...]
        v1c = jnp.concatenate(
            [r[k] for r in v1_refs for k in range(2)], axis=1)
        v2c = jnp.concatenate(
            [r[k] for r in v2_refs for k in range(2)], axis=1)
        w = w_ref[...]
        acc = jnp.dot(tt, w[0], precision=_HI, preferred_element_type=jnp.float32)
        acc = acc - jnp.dot(sv * v1c, w[1], precision=_HI,
                            preferred_element_type=jnp.float32)
        acc = acc + jnp.dot(2.0 * (sv * v2c) - tt, w[2], precision=_HI,
                            preferred_element_type=jnp.float32)
        h = jnp.maximum(acc + b_ref[...], 0.0)
        out_refs[0][...] = h
        if emit_next:
            u = sv * h
            for g in range(ngroups):
                for k in range(2):
                    lo = (2 * g + k) * F2
                    out_refs[1 + g][k] = u[:, lo:lo + F2]

    part_spec = pl.BlockSpec((2, ROWT, F2), lambda i: (0, i, 0))
    in_specs = [pl.BlockSpec((ROWT, Fin), lambda i: (i, 0))]
    in_specs += [part_spec] * (2 * nparts)
    in_specs += [
        pl.BlockSpec((ROWT, 1), lambda i: (i, 0)),
        pl.BlockSpec(W.shape, lambda i: (0, 0, 0)),
        pl.BlockSpec((1, H), lambda i: (0, 0)),
    ]
    out_shape = [jax.ShapeDtypeStruct((npad, H), jnp.float32)]
    out_specs = [pl.BlockSpec((ROWT, H), lambda i: (i, 0))]
    if emit_next:
        for _ in range(ngroups):
            out_shape.append(
                jax.ShapeDtypeStruct((2, npad, F2), jnp.float32))
            out_specs.append(part_spec)

    res = pl.pallas_call(
        body,
        grid=(npad // ROWT,),
        in_specs=in_specs,
        out_specs=out_specs,
        out_shape=out_shape,
    )(t, *v1_parts, *v2_parts, svec, W, b2d)
    return res if emit_next else res[0]


def kernel(x, edge_index, W1, b1, W2, b2):
    N, IN = x.shape
    H = W1.shape[2]
    E = edge_index.shape[1]

    npad = ((N + 2047) // 2048) * 2048
    nblk = 4 * (-(-E // (NSUB * EBLK * 4)))
    ep = NSUB * nblk * EBLK

    row = edge_index[0]
    col = edge_index[1]
    pad = ep - E
    zi = jnp.zeros((pad,), jnp.int32)
    ti = jnp.full((pad,), npad, jnp.int32)
    zx = jnp.zeros((NSUB, 4, EBLK), jnp.int32)

    def _src(a):
        a = jnp.concatenate([a, zi]).reshape(NSUB, nblk, EBLK)
        return jnp.concatenate([a, zx], axis=1)

    src_p = _src(row)
    dst_p = jnp.concatenate([col, ti]).reshape(NSUB, nblk, EBLK)
    src_d = _src(col)
    dst_d = jnp.concatenate([row, ti]).reshape(NSUB, nblk, EBLK)

    xp = jnp.zeros((npad, IN), jnp.float32).at[:N].set(x)

    zbuf = jnp.zeros((EBLK, F2), jnp.float32)
    aprop = _make_aprop(nblk, npad)

    ones_in = jnp.ones((NCORE, npad, F2), jnp.float32)
    degout = aprop(ones_in, src_d, dst_d, zbuf)
    deg = degout[0, :, 0]

    s = jnp.where(deg > 0, jax.lax.rsqrt(jnp.where(deg > 0, deg, 1.0)), 0.0)
    sc = s.reshape(npad, 1)
    s2c = (s * s).reshape(npad, 1)

    u0 = _rowscale_split(xp, sc, npad)
    v1 = aprop(u0, src_p, dst_p, zbuf)
    u1 = _rowscale_stacked(v1, s2c, npad)
    v2 = aprop(u1, src_p, dst_p, zbuf)
    h, uA, uB = _cheb_mix(xp, [v1], [v2], sc, W1, b1, npad, True)

    vA1 = aprop(uA, src_p, dst_p, zbuf)
    vB1 = aprop(uB, src_p, dst_p, zbuf)
    uA1 = _rowscale_stacked(vA1, s2c, npad)
    uB1 = _rowscale_stacked(vB1, s2c, npad)
    vA2 = aprop(uA1, src_p, dst_p, zbuf)
    vB2 = aprop(uB1, src_p, dst_p, zbuf)
    out = _cheb_mix(h, [vA1, vB1], [vA2, vB2], sc, W2, b2, npad, False)

    return out[:N]


# speedup vs baseline: 1.0599x; 1.0599x over previous
import functools

import jax
import jax.numpy as jnp
from jax import lax
from jax.experimental import pallas as pl
from jax.experimental.pallas import tpu as pltpu
from jax.experimental.pallas import tpu_sc as plsc

NSUB = 16
NCORE = 2
EBLK = 128
F2 = 64
ROWT = 256

_HI = jax.lax.Precision.HIGHEST


def _make_aprop(nblk: int, npad: int):
    slab = npad + 16
    rows_per_sub = npad // NSUB
    nchunk = rows_per_sub // 128
    mesh = plsc.VectorSubcoreMesh(core_axis_name="c", subcore_axis_name="s")

    @functools.partial(
        pl.kernel,
        out_type=jax.ShapeDtypeStruct((NCORE, npad, F2), jnp.float32),
        mesh=mesh,
        scratch_types=[
            pltpu.VMEM((nblk + 4, EBLK), jnp.int32),
            pltpu.VMEM((nblk, EBLK), jnp.int32),
            pltpu.VMEM((2, EBLK, F2), jnp.float32),
            pltpu.VMEM_SHARED((slab, F2), jnp.float32),
            pltpu.SemaphoreType.DMA,
            pltpu.SemaphoreType.DMA,
        ],
        compiler_params=pltpu.CompilerParams(use_tc_tiling_on_sc=False),
    )
    def aprop(in_hbm, src_hbm, dst_hbm, zero_hbm, out_hbm,
              src_v, dst_v, gbuf, acc, *sems):
        c = lax.axis_index("c")
        s = lax.axis_index("s")
        pltpu.sync_copy(src_hbm.at[s], src_v)
        pltpu.sync_copy(dst_hbm.at[s], dst_v)
        pltpu.sync_copy(zero_hbm, gbuf.at[0])
        base = s * rows_per_sub
        for k in range(nchunk):
            pltpu.sync_copy(gbuf.at[0], acc.at[pl.ds(base + k * 128, 128)])

        @pl.when(s == NSUB - 1)
        def _():
            pltpu.sync_copy(gbuf.at[0].at[pl.ds(0, 16)], acc.at[pl.ds(npad, 16)])

        plsc.subcore_barrier()

        def run(in_h, out_h):
            pltpu.async_copy(in_h.at[src_v.at[0]], gbuf.at[0], sems[0])

            def body(i, carry):
                j0 = i * 2
                for k in range(2):
                    j = j0 + k
                    q = 1 - k
                    pltpu.make_async_copy(
                        in_h.at[src_v.at[j]], gbuf.at[k], sems[k]).wait()
                    pltpu.async_copy(
                        in_h.at[src_v.at[j + 1]], gbuf.at[q], sems[q])
                    pltpu.sync_copy(gbuf.at[k], acc.at[dst_v.at[j]], add=True)
                return carry

            lax.fori_loop(0, nblk // 2, body, 0)
            pltpu.make_async_copy(
                in_h.at[src_v.at[nblk]], gbuf.at[0], sems[0]).wait()
            plsc.subcore_barrier()
            for k in range(nchunk):
                r = base + k * 128
                pltpu.sync_copy(acc.at[pl.ds(r, 128)], out_h.at[pl.ds(r, 128)])

        @pl.when(c == 0)
        def _():
            run(in_hbm.at[0], out_hbm.at[0])

        @pl.when(c == 1)
        def _():
            run(in_hbm.at[1], out_hbm.at[1])

    return aprop


def _rowscale_split(a, svec, npad):
    F = a.shape[1]

    def body(a_ref, s_ref, o_ref):
        av = a_ref[...] * s_ref[...]
        o_ref[0] = av[:, :F2]
        o_ref[1] = av[:, F2:]

    return pl.pallas_call(
        body,
        grid=(npad // ROWT,),
        in_specs=[
            pl.BlockSpec((ROWT, F), lambda i: (i, 0)),
            pl.BlockSpec((ROWT, 1), lambda i: (i, 0)),
        ],
        out_specs=pl.BlockSpec((2, ROWT, F2), lambda i: (0, i, 0)),
        out_shape=jax.ShapeDtypeStruct((2, npad, F2), jnp.float32),
    )(a, svec)


def _rowscale_stacked(v, svec, npad):
    def body(v_ref, s_ref, o_ref):
        o_ref[...] = v_ref[...] * s_ref[...][None]

    return pl.pallas_call(
        body,
        grid=(npad // ROWT,),
        in_specs=[
            pl.BlockSpec((2, ROWT, F2), lambda i: (0, i, 0)),
            pl.BlockSpec((ROWT, 1), lambda i: (i, 0)),
        ],
        out_specs=pl.BlockSpec((2, ROWT, F2), lambda i: (0, i, 0)),
        out_shape=jax.ShapeDtypeStruct((2, npad, F2), jnp.float32),
    )(v, svec)


def _cheb_mix(t, v1_parts, v2_parts, svec, W, b, npad, emit_next):
    Fin = t.shape[1]
    H = W.shape[2]
    nparts = len(v1_parts)
    ngroups = H // (2 * F2)
    b2d = b.reshape(1, H)

    def body(*refs):
        t_ref = refs[0]
        v1_refs = refs[1:1 + nparts]
        v2_refs = refs[1 + nparts:1 + 2 * nparts]
        s_ref, w_ref, b_ref = refs[1 + 2 * nparts:4 + 2 * nparts]
        out_refs = refs[4 + 2 * nparts:]
        sv = s_ref[...]
        tt = t_ref[...]
        v1c = jnp.concatenate(
            [r[k] for r in v1_refs for k in range(2)], axis=1)
        v2c = jnp.concatenate(
            [r[k] for r in v2_refs for k in range(2)], axis=1)
        w = w_ref[...]
        acc = jnp.dot(tt, w[0], precision=_HI, preferred_element_type=jnp.float32)
        acc = acc - jnp.dot(sv * v1c, w[1], precision=_HI,
                            preferred_element_type=jnp.float32)
        acc = acc + jnp.dot(2.0 * (sv * v2c) - tt, w[2], precision=_HI,
                            preferred_element_type=jnp.float32)
        h = jnp.maximum(acc + b_ref[...], 0.0)
        out_refs[0][...] = h
        if emit_next:
            u = sv * h
            for g in range(ngroups):
                for k in range(2):
                    lo = (2 * g + k) * F2
                    out_refs[1 + g][k] = u[:, lo:lo + F2]

    part_spec = pl.BlockSpec((2, ROWT, F2), lambda i: (0, i, 0))
    in_specs = [pl.BlockSpec((ROWT, Fin), lambda i: (i, 0))]
    in_specs += [part_spec] * (2 * nparts)
    in_specs += [
        pl.BlockSpec((ROWT, 1), lambda i: (i, 0)),
        pl.BlockSpec(W.shape, lambda i: (0, 0, 0)),
        pl.BlockSpec((1, H), lambda i: (0, 0)),
    ]
    out_shape = [jax.ShapeDtypeStruct((npad, H), jnp.float32)]
    out_specs = [pl.BlockSpec((ROWT, H), lambda i: (i, 0))]
    if emit_next:
        for _ in range(ngroups):
            out_shape.append(
                jax.ShapeDtypeStruct((2, npad, F2), jnp.float32))
            out_specs.append(part_spec)

    res = pl.pallas_call(
        body,
        grid=(npad // ROWT,),
        in_specs=in_specs,
        out_specs=out_specs,
        out_shape=out_shape,
    )(t, *v1_parts, *v2_parts, svec, W, b2d)
    return res if emit_next else res[0]


def kernel(x, edge_index, W1, b1, W2, b2):
    N, IN = x.shape
    H = W1.shape[2]
    E = edge_index.shape[1]

    npad = ((N + 2047) // 2048) * 2048
    nblk = 4 * (-(-E // (NSUB * EBLK * 4)))
    ep = NSUB * nblk * EBLK

    row = edge_index[0]
    col = edge_index[1]
    pad = ep - E
    zi = jnp.zeros((pad,), jnp.int32)
    ti = jnp.full((pad,), npad, jnp.int32)
    zx = jnp.zeros((NSUB, 4, EBLK), jnp.int32)

    def _src(a):
        a = jnp.concatenate([a, zi]).reshape(NSUB, nblk, EBLK)
        return jnp.concatenate([a, zx], axis=1)

    src_p = _src(row)
    dst_p = jnp.concatenate([col, ti]).reshape(NSUB, nblk, EBLK)
    src_d = _src(col)
    dst_d = jnp.concatenate([row, ti]).reshape(NSUB, nblk, EBLK)

    xp = jnp.zeros((npad, IN), jnp.float32).at[:N].set(x)

    zbuf = jnp.zeros((EBLK, F2), jnp.float32)
    aprop = _make_aprop(nblk, npad)

    ones_in = jnp.ones((NCORE, npad, F2), jnp.float32)
    degout = aprop(ones_in, src_d, dst_d, zbuf)
    deg = degout[0, :, 0]

    s = jnp.where(deg > 0, jax.lax.rsqrt(jnp.where(deg > 0, deg, 1.0)), 0.0)
    sc = s.reshape(npad, 1)
    s2c = (s * s).reshape(npad, 1)

    u0 = _rowscale_split(xp, sc, npad)
    v1 = aprop(u0, src_p, dst_p, zbuf)
    u1 = _rowscale_stacked(v1, s2c, npad)
    v2 = aprop(u1, src_p, dst_p, zbuf)
    h, uA, uB = _cheb_mix(xp, [v1], [v2], sc, W1, b1, npad, True)

    vA1 = aprop(uA, src_p, dst_p, zbuf)
    vB1 = aprop(uB, src_p, dst_p, zbuf)
    uA1 = _rowscale_stacked(vA1, s2c, npad)
    uB1 = _rowscale_stacked(vB1, s2c, npad)
    vA2 = aprop(uA1, src_p, dst_p, zbuf)
    vB2 = aprop(uB1, src_p, dst_p, zbuf)
    out = _cheb_mix(h, [vA1, vB1], [vA2, vB2], sc, W2, b2, npad, False)

    return out[:N]


# speedup vs baseline: 1.9378x; 1.8282x over previous
import functools

import jax
import jax.numpy as jnp
from jax import lax
from jax.experimental import pallas as pl
from jax.experimental.pallas import tpu as pltpu
from jax.experimental.pallas import tpu_sc as plsc

NSUB = 16
NCORE = 2
EBLK = 128
FW = 128
ROWT = 256

_HI = jax.lax.Precision.HIGHEST


def _make_aprop(nblk: int, npad: int):
    slab = npad
    rows_per_sub = npad // NSUB
    nchunk = rows_per_sub // 128
    mesh = plsc.VectorSubcoreMesh(core_axis_name="c", subcore_axis_name="s")

    @functools.partial(
        pl.kernel,
        out_type=jax.ShapeDtypeStruct((NCORE, npad, FW), jnp.bfloat16),
        mesh=mesh,
        scratch_types=[
            pltpu.VMEM((nblk, EBLK), jnp.int32),
            pltpu.VMEM((nblk, EBLK), jnp.int32),
            pltpu.VMEM((EBLK, FW), jnp.bfloat16),
            pltpu.VMEM_SHARED((slab, FW), jnp.bfloat16),
            pltpu.SemaphoreType.DMA,
        ],
        compiler_params=pltpu.CompilerParams(use_tc_tiling_on_sc=False),
    )
    def aprop(in_hbm, src_hbm, dst_hbm, zero_hbm, out_hbm,
              src_v, dst_v, gbuf, acc, sem):
        c = lax.axis_index("c")
        s = lax.axis_index("s")
        pltpu.sync_copy(src_hbm.at[s], src_v)
        pltpu.sync_copy(dst_hbm.at[s], dst_v)
        base = s * rows_per_sub

        pltpu.sync_copy(zero_hbm, gbuf)
        for k in range(nchunk):
            pltpu.sync_copy(gbuf, acc.at[pl.ds(base + k * 128, 128)])

        plsc.subcore_barrier()

        def run(in_h, out_h):
            def body(j, carry):
                pltpu.async_copy(in_h.at[src_v.at[j]], gbuf, sem).wait()
                pltpu.sync_copy(gbuf, acc.at[dst_v.at[j]], add=True)
                return carry

            lax.fori_loop(0, nblk, body, 0)
            plsc.subcore_barrier()
            for k in range(nchunk):
                r = base + k * 128
                pltpu.sync_copy(acc.at[pl.ds(r, 128)], out_h.at[pl.ds(r, 128)])

        @pl.when(c == 0)
        def _():
            run(in_hbm.at[0], out_hbm.at[0])

        @pl.when(c == 1)
        def _():
            run(in_hbm.at[1], out_hbm.at[1])

    return aprop


def _rowscale_dup(a, svec, npad):
    def body(a_ref, s_ref, o_ref):
        av = (a_ref[...] * s_ref[...]).astype(jnp.bfloat16)
        o_ref[0] = av
        o_ref[1] = av

    return pl.pallas_call(
        body,
        grid=(npad // ROWT,),
        in_specs=[
            pl.BlockSpec((ROWT, FW), lambda i: (i, 0)),
            pl.BlockSpec((ROWT, 1), lambda i: (i, 0)),
        ],
        out_specs=pl.BlockSpec((2, ROWT, FW), lambda i: (0, i, 0)),
        out_shape=jax.ShapeDtypeStruct((2, npad, FW), jnp.bfloat16),
    )(a, svec)


def _rowscale_dup0(v, svec, npad):
    def body(v_ref, s_ref, o_ref):
        av = (v_ref[0].astype(jnp.float32) * s_ref[...]).astype(jnp.bfloat16)
        o_ref[0] = av
        o_ref[1] = av

    return pl.pallas_call(
        body,
        grid=(npad // ROWT,),
        in_specs=[
            pl.BlockSpec((1, ROWT, FW), lambda i: (0, i, 0)),
            pl.BlockSpec((ROWT, 1), lambda i: (i, 0)),
        ],
        out_specs=pl.BlockSpec((2, ROWT, FW), lambda i: (0, i, 0)),
        out_shape=jax.ShapeDtypeStruct((2, npad, FW), jnp.bfloat16),
    )(v, svec)


def _rowscale_stacked(v, svec, npad):
    def body(v_ref, s_ref, o_ref):
        o_ref[...] = (v_ref[...].astype(jnp.float32)
                      * s_ref[...][None]).astype(jnp.bfloat16)

    return pl.pallas_call(
        body,
        grid=(npad // ROWT,),
        in_specs=[
            pl.BlockSpec((2, ROWT, FW), lambda i: (0, i, 0)),
            pl.BlockSpec((ROWT, 1), lambda i: (i, 0)),
        ],
        out_specs=pl.BlockSpec((2, ROWT, FW), lambda i: (0, i, 0)),
        out_shape=jax.ShapeDtypeStruct((2, npad, FW), jnp.bfloat16),
    )(v, svec)


def _cheb_mix(t, v1, v2, svec, W, b, npad, split_v, emit_next):
    Fin = t.shape[1]
    H = W.shape[2]
    H2 = H // 2
    b2d = b.reshape(1, H)

    def body(t_ref, v1_ref, v2_ref, s_ref, w_ref, b_ref, *out_refs):
        sv = s_ref[...]
        tt = t_ref[...]
        if split_v:
            v1c = jnp.concatenate([v1_ref[0], v1_ref[1]], axis=1)
            v2c = jnp.concatenate([v2_ref[0], v2_ref[1]], axis=1)
        else:
            v1c = v1_ref[0]
            v2c = v2_ref[0]
        v1c = v1c.astype(jnp.float32)
        v2c = v2c.astype(jnp.float32)
        w = w_ref[...]
        acc = jnp.dot(tt, w[0], precision=_HI, preferred_element_type=jnp.float32)
        acc = acc - jnp.dot(sv * v1c, w[1], precision=_HI,
                            preferred_element_type=jnp.float32)
        acc = acc + jnp.dot(2.0 * (sv * v2c) - tt, w[2], precision=_HI,
                            preferred_element_type=jnp.float32)
        h = jnp.maximum(acc + b_ref[...], 0.0)
        out_refs[0][...] = h
        if emit_next:
            u = (sv * h).astype(jnp.bfloat16)
            out_refs[1][0] = u[:, :H2]
            out_refs[1][1] = u[:, H2:]

    nv = 2 if split_v else 1
    in_specs = [
        pl.BlockSpec((ROWT, Fin), lambda i: (i, 0)),
        pl.BlockSpec((nv, ROWT, FW), lambda i: (0, i, 0)),
        pl.BlockSpec((nv, ROWT, FW), lambda i: (0, i, 0)),
        pl.BlockSpec((ROWT, 1), lambda i: (i, 0)),
        pl.BlockSpec(W.shape, lambda i: (0, 0, 0)),
        pl.BlockSpec((1, H), lambda i: (0, 0)),
    ]
    out_shape = [jax.ShapeDtypeStruct((npad, H), jnp.float32)]
    out_specs = [pl.BlockSpec((ROWT, H), lambda i: (i, 0))]
    if emit_next:
        out_shape.append(jax.ShapeDtypeStruct((2, npad, H2), jnp.bfloat16))
        out_specs.append(pl.BlockSpec((2, ROWT, H2), lambda i: (0, i, 0)))

    res = pl.pallas_call(
        body,
        grid=(npad // ROWT,),
        in_specs=in_specs,
        out_specs=out_specs,
        out_shape=out_shape,
    )(t, v1, v2, svec, W, b2d)
    return res if emit_next else res[0]


def kernel(x, edge_index, W1, b1, W2, b2):
    N, IN = x.shape
    H = W1.shape[2]
    E = edge_index.shape[1]

    npad = ((N + 2048) // 2048) * 2048
    nblk = -(-E // (NSUB * EBLK))
    ep = NSUB * nblk * EBLK

    row = edge_index[0]
    col = edge_index[1]
    pad = ep - E
    zi = jnp.zeros((pad,), jnp.int32)
    ti = jnp.full((pad,), npad - 1, jnp.int32)
    src_p = jnp.concatenate([row, zi]).reshape(NSUB, nblk, EBLK)
    dst_p = jnp.concatenate([col, ti]).reshape(NSUB, nblk, EBLK)
    src_d = jnp.concatenate([col, zi]).reshape(NSUB, nblk, EBLK)
    dst_d = jnp.concatenate([row, ti]).reshape(NSUB, nblk, EBLK)

    xp = jnp.zeros((npad, IN), jnp.float32).at[:N].set(x)

    zbuf = jnp.zeros((EBLK, FW), jnp.bfloat16)
    aprop = _make_aprop(nblk, npad)

    ones_in = jnp.ones((NCORE, npad, FW), jnp.bfloat16)
    degout = aprop(ones_in, src_d, dst_d, zbuf)
    deg = degout[0, :, 0].astype(jnp.float32)

    s = jnp.where(deg > 0, jax.lax.rsqrt(jnp.where(deg > 0, deg, 1.0)), 0.0)
    sc = s.reshape(npad, 1)
    s2c = (s * s).reshape(npad, 1)

    u0 = _rowscale_dup(xp, sc, npad)
    v1 = aprop(u0, src_p, dst_p, zbuf)
    u1 = _rowscale_dup0(v1, s2c, npad)
    v2 = aprop(u1, src_p, dst_p, zbuf)
    h, u0b = _cheb_mix(xp, v1, v2, sc, W1, b1, npad, False, True)

    v1b = aprop(u0b, src_p, dst_p, zbuf)
    u1b = _rowscale_stacked(v1b, s2c, npad)
    v2b = aprop(u1b, src_p, dst_p, zbuf)
    out = _cheb_mix(h, v1b, v2b, sc, W2, b2, npad, True, False)

    return out[:N]


# speedup vs baseline: 2.3252x; 1.2000x over previous
import functools

import jax
import jax.numpy as jnp
from jax import lax
from jax.experimental import pallas as pl
from jax.experimental.pallas import tpu as pltpu
from jax.experimental.pallas import tpu_sc as plsc

NSUB = 16
NCORE = 2
EBLK = 128
FW = 128
ROWT = 256

_HI = jax.lax.Precision.HIGHEST


def _make_aprop(nblk: int, npad: int):
    slab = npad
    rows_per_sub = npad // NSUB
    nchunk = rows_per_sub // 128
    mesh = plsc.VectorSubcoreMesh(core_axis_name="c", subcore_axis_name="s")

    @functools.partial(
        pl.kernel,
        out_type=jax.ShapeDtypeStruct((NCORE, npad, FW), jnp.bfloat16),
        mesh=mesh,
        scratch_types=[
            pltpu.VMEM((nblk, EBLK), jnp.int32),
            pltpu.VMEM((nblk, EBLK), jnp.int32),
            pltpu.VMEM((EBLK, FW), jnp.bfloat16),
            pltpu.VMEM_SHARED((slab, FW), jnp.bfloat16),
            pltpu.SemaphoreType.DMA,
        ],
        compiler_params=pltpu.CompilerParams(use_tc_tiling_on_sc=False),
    )
    def aprop(in_hbm, src_hbm, dst_hbm, zero_hbm, out_hbm,
              src_v, dst_v, gbuf, acc, sem):
        c = lax.axis_index("c")
        s = lax.axis_index("s")
        pltpu.sync_copy(src_hbm.at[s], src_v)
        pltpu.sync_copy(dst_hbm.at[s], dst_v)
        base = s * rows_per_sub

        pltpu.sync_copy(zero_hbm, gbuf)
        for k in range(nchunk):
            pltpu.sync_copy(gbuf, acc.at[pl.ds(base + k * 128, 128)])

        plsc.subcore_barrier()

        def run(in_h, out_h):
            def body(j, carry):
                pltpu.async_copy(in_h.at[src_v.at[j]], gbuf, sem).wait()
                pltpu.sync_copy(gbuf, acc.at[dst_v.at[j]], add=True)
                return carry

            lax.fori_loop(0, nblk, body, 0)
            plsc.subcore_barrier()
            for k in range(nchunk):
                r = base + k * 128
                pltpu.sync_copy(acc.at[pl.ds(r, 128)], out_h.at[pl.ds(r, 128)])

        @pl.when(c == 0)
        def _():
            run(in_hbm.at[0], out_hbm.at[0])

        @pl.when(c == 1)
        def _():
            run(in_hbm.at[1], out_hbm.at[1])

    return aprop


def _make_deghist(e32: int):
    niter = e32 // 16
    mesh = plsc.VectorSubcoreMesh(core_axis_name="c", subcore_axis_name="s")

    @functools.partial(
        pl.kernel,
        out_type=jax.ShapeDtypeStruct((NCORE, 128, 128), jnp.float32),
        mesh=mesh,
        scratch_types=[
            pltpu.VMEM((e32,), jnp.int32),
            pltpu.VMEM((128, 128), jnp.float32),
            pltpu.VMEM((1, 128), jnp.int32),
            pltpu.VMEM_SHARED((128, 128), jnp.float32),
        ],
        compiler_params=pltpu.CompilerParams(use_tc_tiling_on_sc=False,
                                             needs_layout_passes=False),
    )
    def deghist(rows_hbm, id_hbm, zero_hbm, out_hbm, idxv, hist, idv, spdeg):
        c = lax.axis_index("c")
        s = lax.axis_index("s")
        w = c * NSUB + s
        pltpu.sync_copy(rows_hbm.at[w], idxv)
        pltpu.sync_copy(id_hbm, idv)
        pltpu.sync_copy(zero_hbm, hist)
        pltpu.sync_copy(zero_hbm.at[pl.ds(0, 8)], spdeg.at[pl.ds(s * 8, 8)])
        ones = jnp.ones((16,), jnp.float32)

        def body(i, carry):
            ix = idxv[pl.ds(i * 16, 16)]
            plsc.addupdate_scatter(
                hist,
                [lax.shift_right_logical(ix, 7), lax.bitwise_and(ix, 127)],
                ones)
            return carry

        lax.fori_loop(0, niter, body, 0)
        plsc.subcore_barrier()
        pltpu.sync_copy(hist, spdeg.at[idv.at[0]], add=True)
        plsc.subcore_barrier()

        @pl.when(c == 0)
        def _():
            pltpu.sync_copy(spdeg.at[pl.ds(s * 8, 8)],
                            out_hbm.at[0].at[pl.ds(s * 8, 8)])

        @pl.when(c == 1)
        def _():
            pltpu.sync_copy(spdeg.at[pl.ds(s * 8, 8)],
                            out_hbm.at[1].at[pl.ds(s * 8, 8)])

    return deghist


def _rowscale_dup(a, svec, npad):
    def body(a_ref, s_ref, o_ref):
        av = (a_ref[...] * s_ref[...]).astype(jnp.bfloat16)
        o_ref[0] = av
        o_ref[1] = av

    return pl.pallas_call(
        body,
        grid=(npad // ROWT,),
        in_specs=[
            pl.BlockSpec((ROWT, FW), lambda i: (i, 0)),
            pl.BlockSpec((ROWT, 1), lambda i: (i, 0)),
        ],
        out_specs=pl.BlockSpec((2, ROWT, FW), lambda i: (0, i, 0)),
        out_shape=jax.ShapeDtypeStruct((2, npad, FW), jnp.bfloat16),
    )(a, svec)


def _rowscale_dup0(v, svec, npad):
    def body(v_ref, s_ref, o_ref):
        av = (v_ref[0].astype(jnp.float32) * s_ref[...]).astype(jnp.bfloat16)
        o_ref[0] = av
        o_ref[1] = av

    return pl.pallas_call(
        body,
        grid=(npad // ROWT,),
        in_specs=[
            pl.BlockSpec((1, ROWT, FW), lambda i: (0, i, 0)),
            pl.BlockSpec((ROWT, 1), lambda i: (i, 0)),
        ],
        out_specs=pl.BlockSpec((2, ROWT, FW), lambda i: (0, i, 0)),
        out_shape=jax.ShapeDtypeStruct((2, npad, FW), jnp.bfloat16),
    )(v, svec)


def _rowscale_stacked(v, svec, npad):
    def body(v_ref, s_ref, o_ref):
        o_ref[...] = (v_ref[...].astype(jnp.float32)
                      * s_ref[...][None]).astype(jnp.bfloat16)

    return pl.pallas_call(
        body,
        grid=(npad // ROWT,),
        in_specs=[
            pl.BlockSpec((2, ROWT, FW), lambda i: (0, i, 0)),
            pl.BlockSpec((ROWT, 1), lambda i: (i, 0)),
        ],
        out_specs=pl.BlockSpec((2, ROWT, FW), lambda i: (0, i, 0)),
        out_shape=jax.ShapeDtypeStruct((2, npad, FW), jnp.bfloat16),
    )(v, svec)


def _cheb_mix(t, v1, v2, svec, W, b, npad, split_v, emit_next):
    Fin = t.shape[1]
    H = W.shape[2]
    H2 = H // 2
    b2d = b.reshape(1, H)

    def body(t_ref, v1_ref, v2_ref, s_ref, w_ref, b_ref, *out_refs):
        sv = s_ref[...]
        tt = t_ref[...]
        if split_v:
            v1c = jnp.concatenate([v1_ref[0], v1_ref[1]], axis=1)
            v2c = jnp.concatenate([v2_ref[0], v2_ref[1]], axis=1)
        else:
            v1c = v1_ref[0]
            v2c = v2_ref[0]
        v1c = v1c.astype(jnp.float32)
        v2c = v2c.astype(jnp.float32)
        w = w_ref[...]
        acc = jnp.dot(tt, w[0], precision=_HI, preferred_element_type=jnp.float32)
        acc = acc - jnp.dot(sv * v1c, w[1], precision=_HI,
                            preferred_element_type=jnp.float32)
        acc = acc + jnp.dot(2.0 * (sv * v2c) - tt, w[2], precision=_HI,
                            preferred_element_type=jnp.float32)
        h = jnp.maximum(acc + b_ref[...], 0.0)
        out_refs[0][...] = h
        if emit_next:
            u = (sv * h).astype(jnp.bfloat16)
            out_refs[1][0] = u[:, :H2]
            out_refs[1][1] = u[:, H2:]

    nv = 2 if split_v else 1
    in_specs = [
        pl.BlockSpec((ROWT, Fin), lambda i: (i, 0)),
        pl.BlockSpec((nv, ROWT, FW), lambda i: (0, i, 0)),
        pl.BlockSpec((nv, ROWT, FW), lambda i: (0, i, 0)),
        pl.BlockSpec((ROWT, 1), lambda i: (i, 0)),
        pl.BlockSpec(W.shape, lambda i: (0, 0, 0)),
        pl.BlockSpec((1, H), lambda i: (0, 0)),
    ]
    out_shape = [jax.ShapeDtypeStruct((npad, H), jnp.float32)]
    out_specs = [pl.BlockSpec((ROWT, H), lambda i: (i, 0))]
    if emit_next:
        out_shape.append(jax.ShapeDtypeStruct((2, npad, H2), jnp.bfloat16))
        out_specs.append(pl.BlockSpec((2, ROWT, H2), lambda i: (0, i, 0)))

    res = pl.pallas_call(
        body,
        grid=(npad // ROWT,),
        in_specs=in_specs,
        out_specs=out_specs,
        out_shape=out_shape,
    )(t, v1, v2, svec, W, b2d)
    return res if emit_next else res[0]


def kernel(x, edge_index, W1, b1, W2, b2):
    N, IN = x.shape
    H = W1.shape[2]
    E = edge_index.shape[1]

    npad = ((N + 2048) // 2048) * 2048
    nblk = -(-E // (NSUB * EBLK))
    ep = NSUB * nblk * EBLK

    row = edge_index[0]
    col = edge_index[1]
    pad = ep - E
    zi = jnp.zeros((pad,), jnp.int32)
    ti = jnp.full((pad,), npad - 1, jnp.int32)
    src_p = jnp.concatenate([row, zi]).reshape(NSUB, nblk, EBLK)
    dst_p = jnp.concatenate([col, ti]).reshape(NSUB, nblk, EBLK)
    src_d = jnp.concatenate([col, zi]).reshape(NSUB, nblk, EBLK)
    dst_d = jnp.concatenate([row, ti]).reshape(NSUB, nblk, EBLK)

    xp = jnp.zeros((npad, IN), jnp.float32).at[:N].set(x)

    zbuf = jnp.zeros((EBLK, FW), jnp.bfloat16)
    aprop = _make_aprop(nblk, npad)

    ep32 = 512 * (-(-E // 512))
    rows32 = jnp.concatenate(
        [row, jnp.full((ep32 - E,), 128 * 128 - 1, jnp.int32)]).reshape(32, -1)
    ididx = jnp.arange(128, dtype=jnp.int32).reshape(1, 128)
    zf = jnp.zeros((128, 128), jnp.float32)
    degq = _make_deghist(ep32 // 32)(rows32, ididx, zf)
    deg = (degq[0] + degq[1]).reshape(-1)[:npad]

    s = jnp.where(deg > 0, jax.lax.rsqrt(jnp.where(deg > 0, deg, 1.0)), 0.0)
    sc = s.reshape(npad, 1)
    s2c = (s * s).reshape(npad, 1)

    u0 = _rowscale_dup(xp, sc, npad)
    v1 = aprop(u0, src_p, dst_p, zbuf)
    u1 = _rowscale_dup0(v1, s2c, npad)
    v2 = aprop(u1, src_p, dst_p, zbuf)
    h, u0b = _cheb_mix(xp, v1, v2, sc, W1, b1, npad, False, True)

    v1b = aprop(u0b, src_p, dst_p, zbuf)
    u1b = _rowscale_stacked(v1b, s2c, npad)
    v2b = aprop(u1b, src_p, dst_p, zbuf)
    out = _cheb_mix(h, v1b, v2b, sc, W2, b2, npad, True, False)

    return out[:N]


# speedup vs baseline: 2.3740x; 1.0210x over previous
import functools

import jax
import jax.numpy as jnp
from jax import lax
from jax.experimental import pallas as pl
from jax.experimental.pallas import tpu as pltpu
from jax.experimental.pallas import tpu_sc as plsc

NSUB = 16
NCORE = 2
EBLK = 128
FW = 128
ROWT = 256

_HI = jax.lax.Precision.DEFAULT


def _make_aprop(nblk: int, npad: int):
    slab = npad
    rows_per_sub = npad // NSUB
    nchunk = rows_per_sub // 128
    mesh = plsc.VectorSubcoreMesh(core_axis_name="c", subcore_axis_name="s")

    @functools.partial(
        pl.kernel,
        out_type=jax.ShapeDtypeStruct((NCORE, npad, FW), jnp.bfloat16),
        mesh=mesh,
        scratch_types=[
            pltpu.VMEM((nblk, EBLK), jnp.int32),
            pltpu.VMEM((nblk, EBLK), jnp.int32),
            pltpu.VMEM((EBLK, FW), jnp.bfloat16),
            pltpu.VMEM_SHARED((slab, FW), jnp.bfloat16),
            pltpu.SemaphoreType.DMA,
        ],
        compiler_params=pltpu.CompilerParams(use_tc_tiling_on_sc=False),
    )
    def aprop(in_hbm, src_hbm, dst_hbm, zero_hbm, out_hbm,
              src_v, dst_v, gbuf, acc, sem):
        c = lax.axis_index("c")
        s = lax.axis_index("s")
        pltpu.sync_copy(src_hbm.at[s], src_v)
        pltpu.sync_copy(dst_hbm.at[s], dst_v)
        base = s * rows_per_sub

        pltpu.sync_copy(zero_hbm, gbuf)
        for k in range(nchunk):
            pltpu.sync_copy(gbuf, acc.at[pl.ds(base + k * 128, 128)])

        plsc.subcore_barrier()

        def run(in_h, out_h):
            def body(j, carry):
                pltpu.async_copy(in_h.at[src_v.at[j]], gbuf, sem).wait()
                pltpu.sync_copy(gbuf, acc.at[dst_v.at[j]], add=True)
                return carry

            lax.fori_loop(0, nblk, body, 0)
            plsc.subcore_barrier()
            for k in range(nchunk):
                r = base + k * 128
                pltpu.sync_copy(acc.at[pl.ds(r, 128)], out_h.at[pl.ds(r, 128)])

        @pl.when(c == 0)
        def _():
            run(in_hbm.at[0], out_hbm.at[0])

        @pl.when(c == 1)
        def _():
            run(in_hbm.at[1], out_hbm.at[1])

    return aprop


def _make_deghist(e32: int):
    niter = e32 // 16
    mesh = plsc.VectorSubcoreMesh(core_axis_name="c", subcore_axis_name="s")

    @functools.partial(
        pl.kernel,
        out_type=jax.ShapeDtypeStruct((NCORE, 128, 128), jnp.float32),
        mesh=mesh,
        scratch_types=[
            pltpu.VMEM((e32,), jnp.int32),
            pltpu.VMEM((128, 128), jnp.float32),
            pltpu.VMEM((1, 128), jnp.int32),
            pltpu.VMEM_SHARED((128, 128), jnp.float32),
        ],
        compiler_params=pltpu.CompilerParams(use_tc_tiling_on_sc=False,
                                             needs_layout_passes=False),
    )
    def deghist(rows_hbm, id_hbm, zero_hbm, out_hbm, idxv, hist, idv, spdeg):
        c = lax.axis_index("c")
        s = lax.axis_index("s")
        w = c * NSUB + s
        pltpu.sync_copy(rows_hbm.at[w], idxv)
        pltpu.sync_copy(id_hbm, idv)
        pltpu.sync_copy(zero_hbm, hist)
        pltpu.sync_copy(zero_hbm.at[pl.ds(0, 8)], spdeg.at[pl.ds(s * 8, 8)])
        ones = jnp.ones((16,), jnp.float32)

        def body(i, carry):
            ix = idxv[pl.ds(i * 16, 16)]
            plsc.addupdate_scatter(
                hist,
                [lax.shift_right_logical(ix, 7), lax.bitwise_and(ix, 127)],
                ones)
            return carry

        lax.fori_loop(0, niter, body, 0)
        plsc.subcore_barrier()
        pltpu.sync_copy(hist, spdeg.at[idv.at[0]], add=True)
        plsc.subcore_barrier()

        @pl.when(c == 0)
        def _():
            pltpu.sync_copy(spdeg.at[pl.ds(s * 8, 8)],
                            out_hbm.at[0].at[pl.ds(s * 8, 8)])

        @pl.when(c == 1)
        def _():
            pltpu.sync_copy(spdeg.at[pl.ds(s * 8, 8)],
                            out_hbm.at[1].at[pl.ds(s * 8, 8)])

    return deghist


def _rowscale_dup(a, svec, npad):
    def body(a_ref, s_ref, o_ref):
        av = (a_ref[...] * s_ref[...]).astype(jnp.bfloat16)
        o_ref[0] = av
        o_ref[1] = av

    return pl.pallas_call(
        body,
        grid=(npad // ROWT,),
        in_specs=[
            pl.BlockSpec((ROWT, FW), lambda i: (i, 0)),
            pl.BlockSpec((ROWT, 1), lambda i: (i, 0)),
        ],
        out_specs=pl.BlockSpec((2, ROWT, FW), lambda i: (0, i, 0)),
        out_shape=jax.ShapeDtypeStruct((2, npad, FW), jnp.bfloat16),
    )(a, svec)


def _rowscale_dup0(v, svec, npad):
    def body(v_ref, s_ref, o_ref):
        av = (v_ref[0].astype(jnp.float32) * s_ref[...]).astype(jnp.bfloat16)
        o_ref[0] = av
        o_ref[1] = av

    return pl.pallas_call(
        body,
        grid=(npad // ROWT,),
        in_specs=[
            pl.BlockSpec((1, ROWT, FW), lambda i: (0, i, 0)),
            pl.BlockSpec((ROWT, 1), lambda i: (i, 0)),
        ],
        out_specs=pl.BlockSpec((2, ROWT, FW), lambda i: (0, i, 0)),
        out_shape=jax.ShapeDtypeStruct((2, npad, FW), jnp.bfloat16),
    )(v, svec)


def _rowscale_stacked(v, svec, npad):
    def body(v_ref, s_ref, o_ref):
        o_ref[...] = (v_ref[...].astype(jnp.float32)
                      * s_ref[...][None]).astype(jnp.bfloat16)

    return pl.pallas_call(
        body,
        grid=(npad // ROWT,),
        in_specs=[
            pl.BlockSpec((2, ROWT, FW), lambda i: (0, i, 0)),
            pl.BlockSpec((ROWT, 1), lambda i: (i, 0)),
        ],
        out_specs=pl.BlockSpec((2, ROWT, FW), lambda i: (0, i, 0)),
        out_shape=jax.ShapeDtypeStruct((2, npad, FW), jnp.bfloat16),
    )(v, svec)


def _cheb_mix(t, v1, v2, svec, W, b, npad, split_v, emit_next):
    Fin = t.shape[1]
    H = W.shape[2]
    H2 = H // 2
    b2d = b.reshape(1, H)

    def body(t_ref, v1_ref, v2_ref, s_ref, w_ref, b_ref, *out_refs):
        sv = s_ref[...]
        tt = t_ref[...]
        if split_v:
            v1c = jnp.concatenate([v1_ref[0], v1_ref[1]], axis=1)
            v2c = jnp.concatenate([v2_ref[0], v2_ref[1]], axis=1)
        else:
            v1c = v1_ref[0]
            v2c = v2_ref[0]
        v1c = v1c.astype(jnp.float32)
        v2c = v2c.astype(jnp.float32)
        w = w_ref[...].astype(jnp.bfloat16)
        tb = tt.astype(jnp.bfloat16)
        acc = jnp.dot(tb, w[0], precision=_HI, preferred_element_type=jnp.float32)
        acc = acc - jnp.dot((sv * v1c).astype(jnp.bfloat16), w[1], precision=_HI,
                            preferred_element_type=jnp.float32)
        acc = acc + jnp.dot((2.0 * (sv * v2c) - tt).astype(jnp.bfloat16), w[2],
                            precision=_HI, preferred_element_type=jnp.float32)
        h = jnp.maximum(acc + b_ref[...], 0.0)
        out_refs[0][...] = h
        if emit_next:
            u = (sv * h).astype(jnp.bfloat16)
            out_refs[1][0] = u[:, :H2]
            out_refs[1][1] = u[:, H2:]

    nv = 2 if split_v else 1
    in_specs = [
        pl.BlockSpec((ROWT, Fin), lambda i: (i, 0)),
        pl.BlockSpec((nv, ROWT, FW), lambda i: (0, i, 0)),
        pl.BlockSpec((nv, ROWT, FW), lambda i: (0, i, 0)),
        pl.BlockSpec((ROWT, 1), lambda i: (i, 0)),
        pl.BlockSpec(W.shape, lambda i: (0, 0, 0)),
        pl.BlockSpec((1, H), lambda i: (0, 0)),
    ]
    out_shape = [jax.ShapeDtypeStruct((npad, H), jnp.float32)]
    out_specs = [pl.BlockSpec((ROWT, H), lambda i: (i, 0))]
    if emit_next:
        out_shape.append(jax.ShapeDtypeStruct((2, npad, H2), jnp.bfloat16))
        out_specs.append(pl.BlockSpec((2, ROWT, H2), lambda i: (0, i, 0)))

    res = pl.pallas_call(
        body,
        grid=(npad // ROWT,),
        in_specs=in_specs,
        out_specs=out_specs,
        out_shape=out_shape,
    )(t, v1, v2, svec, W, b2d)
    return res if emit_next else res[0]


def kernel(x, edge_index, W1, b1, W2, b2):
    N, IN = x.shape
    H = W1.shape[2]
    E = edge_index.shape[1]

    npad = ((N + 2048) // 2048) * 2048
    nblk = -(-E // (NSUB * EBLK))
    ep = NSUB * nblk * EBLK

    row = edge_index[0]
    col = edge_index[1]
    pad = ep - E
    zi = jnp.zeros((pad,), jnp.int32)
    ti = jnp.full((pad,), npad - 1, jnp.int32)
    src_p = jnp.concatenate([row, zi]).reshape(NSUB, nblk, EBLK)
    dst_p = jnp.concatenate([col, ti]).reshape(NSUB, nblk, EBLK)
    src_d = jnp.concatenate([col, zi]).reshape(NSUB, nblk, EBLK)
    dst_d = jnp.concatenate([row, ti]).reshape(NSUB, nblk, EBLK)

    xp = jnp.zeros((npad, IN), jnp.float32).at[:N].set(x)

    zbuf = jnp.zeros((EBLK, FW), jnp.bfloat16)
    aprop = _make_aprop(nblk, npad)

    ep32 = 512 * (-(-E // 512))
    rows32 = jnp.concatenate(
        [row, jnp.full((ep32 - E,), 128 * 128 - 1, jnp.int32)]).reshape(32, -1)
    ididx = jnp.arange(128, dtype=jnp.int32).reshape(1, 128)
    zf = jnp.zeros((128, 128), jnp.float32)
    degq = _make_deghist(ep32 // 32)(rows32, ididx, zf)
    deg = (degq[0] + degq[1]).reshape(-1)[:npad]

    s = jnp.where(deg > 0, jax.lax.rsqrt(jnp.where(deg > 0, deg, 1.0)), 0.0)
    sc = s.reshape(npad, 1)
    s2c = (s * s).reshape(npad, 1)

    u0 = _rowscale_dup(xp, sc, npad)
    v1 = aprop(u0, src_p, dst_p, zbuf)
    u1 = _rowscale_dup0(v1, s2c, npad)
    v2 = aprop(u1, src_p, dst_p, zbuf)
    h, u0b = _cheb_mix(xp, v1, v2, sc, W1, b1, npad, False, True)

    v1b = aprop(u0b, src_p, dst_p, zbuf)
    u1b = _rowscale_stacked(v1b, s2c, npad)
    v2b = aprop(u1b, src_p, dst_p, zbuf)
    out = _cheb_mix(h, v1b, v2b, sc, W2, b2, npad, True, False)

    return out[:N]


# speedup vs baseline: 2.3747x; 1.0003x over previous
import functools

import jax
import jax.numpy as jnp
from jax import lax
from jax.experimental import pallas as pl
from jax.experimental.pallas import tpu as pltpu
from jax.experimental.pallas import tpu_sc as plsc

NSUB = 16
NCORE = 2
EBLK = 128
FW = 128
ROWT = 256

_HI = jax.lax.Precision.DEFAULT


def _make_aprop(nblk: int, npad: int):
    slab = npad
    rows_per_sub = npad // NSUB
    nchunk = rows_per_sub // 128
    mesh = plsc.VectorSubcoreMesh(core_axis_name="c", subcore_axis_name="s")

    @functools.partial(
        pl.kernel,
        out_type=jax.ShapeDtypeStruct((NCORE, npad, FW), jnp.bfloat16),
        mesh=mesh,
        scratch_types=[
            pltpu.VMEM((nblk, EBLK), jnp.int32),
            pltpu.VMEM((nblk, EBLK), jnp.int32),
            pltpu.VMEM((EBLK, FW), jnp.bfloat16),
            pltpu.VMEM_SHARED((slab, FW), jnp.bfloat16),
            pltpu.SemaphoreType.DMA,
        ],
        compiler_params=pltpu.CompilerParams(use_tc_tiling_on_sc=False),
    )
    def aprop(in_hbm, src_hbm, dst_hbm, zero_hbm, out_hbm,
              src_v, dst_v, gbuf, acc, sem):
        c = lax.axis_index("c")
        s = lax.axis_index("s")
        pltpu.sync_copy(src_hbm.at[s], src_v)
        pltpu.sync_copy(dst_hbm.at[s], dst_v)
        base = s * rows_per_sub

        pltpu.sync_copy(zero_hbm, gbuf)
        for k in range(nchunk):
            pltpu.sync_copy(gbuf, acc.at[pl.ds(base + k * 128, 128)])

        plsc.subcore_barrier()

        def run(in_h, out_h):
            def body(j, carry):
                pltpu.async_copy(in_h.at[src_v.at[j]], gbuf, sem).wait()
                pltpu.sync_copy(gbuf, acc.at[dst_v.at[j]], add=True)
                return carry

            lax.fori_loop(0, nblk, body, 0)
            plsc.subcore_barrier()
            for k in range(nchunk):
                r = base + k * 128
                pltpu.sync_copy(acc.at[pl.ds(r, 128)], out_h.at[pl.ds(r, 128)])

        @pl.when(c == 0)
        def _():
            run(in_hbm.at[0], out_hbm.at[0])

        @pl.when(c == 1)
        def _():
            run(in_hbm.at[1], out_hbm.at[1])

    return aprop


def _make_deghist(e32: int):
    niter = e32 // 16
    mesh = plsc.VectorSubcoreMesh(core_axis_name="c", subcore_axis_name="s")

    @functools.partial(
        pl.kernel,
        out_type=jax.ShapeDtypeStruct((NCORE, 128, 128), jnp.float32),
        mesh=mesh,
        scratch_types=[
            pltpu.VMEM((e32,), jnp.int32),
            pltpu.VMEM((128, 128), jnp.float32),
            pltpu.VMEM((1, 128), jnp.int32),
            pltpu.VMEM_SHARED((128, 128), jnp.float32),
        ],
        compiler_params=pltpu.CompilerParams(use_tc_tiling_on_sc=False,
                                             needs_layout_passes=False),
    )
    def deghist(rows_hbm, id_hbm, zero_hbm, out_hbm, idxv, hist, idv, spdeg):
        c = lax.axis_index("c")
        s = lax.axis_index("s")
        w = c * NSUB + s
        pltpu.sync_copy(rows_hbm.at[w], idxv)
        pltpu.sync_copy(id_hbm, idv)
        pltpu.sync_copy(zero_hbm, hist)
        pltpu.sync_copy(zero_hbm.at[pl.ds(0, 8)], spdeg.at[pl.ds(s * 8, 8)])
        ones = jnp.ones((16,), jnp.float32)

        def body(i, carry):
            ix = idxv[pl.ds(i * 16, 16)]
            plsc.addupdate_scatter(
                hist,
                [lax.shift_right_logical(ix, 7), lax.bitwise_and(ix, 127)],
                ones)
            return carry

        lax.fori_loop(0, niter, body, 0)
        plsc.subcore_barrier()
        pltpu.sync_copy(hist, spdeg.at[idv.at[0]], add=True)
        plsc.subcore_barrier()

        @pl.when(c == 0)
        def _():
            pltpu.sync_copy(spdeg.at[pl.ds(s * 8, 8)],
                            out_hbm.at[0].at[pl.ds(s * 8, 8)])

        @pl.when(c == 1)
        def _():
            pltpu.sync_copy(spdeg.at[pl.ds(s * 8, 8)],
                            out_hbm.at[1].at[pl.ds(s * 8, 8)])

    return deghist


def _rowscale_dup(a, svec, npad):
    def body(a_ref, s_ref, o_ref):
        av = (a_ref[...] * s_ref[...]).astype(jnp.bfloat16)
        o_ref[0] = av
        o_ref[1] = av

    return pl.pallas_call(
        body,
        grid=(npad // ROWT,),
        in_specs=[
            pl.BlockSpec((ROWT, FW), lambda i: (i, 0)),
            pl.BlockSpec((ROWT, 1), lambda i: (i, 0)),
        ],
        out_specs=pl.BlockSpec((2, ROWT, FW), lambda i: (0, i, 0)),
        out_shape=jax.ShapeDtypeStruct((2, npad, FW), jnp.bfloat16),
    )(a, svec)


def _rowscale_dup0(v, svec, npad):
    def body(v_ref, s_ref, o_ref):
        av = (v_ref[0].astype(jnp.float32) * s_ref[...]).astype(jnp.bfloat16)
        o_ref[0] = av
        o_ref[1] = av

    return pl.pallas_call(
        body,
        grid=(npad // ROWT,),
        in_specs=[
            pl.BlockSpec((1, ROWT, FW), lambda i: (0, i, 0)),
            pl.BlockSpec((ROWT, 1), lambda i: (i, 0)),
        ],
        out_specs=pl.BlockSpec((2, ROWT, FW), lambda i: (0, i, 0)),
        out_shape=jax.ShapeDtypeStruct((2, npad, FW), jnp.bfloat16),
    )(v, svec)


def _rowscale_stacked(v, svec, npad):
    def body(v_ref, s_ref, o_ref):
        o_ref[...] = (v_ref[...].astype(jnp.float32)
                      * s_ref[...][None]).astype(jnp.bfloat16)

    return pl.pallas_call(
        body,
        grid=(npad // ROWT,),
        in_specs=[
            pl.BlockSpec((2, ROWT, FW), lambda i: (0, i, 0)),
            pl.BlockSpec((ROWT, 1), lambda i: (i, 0)),
        ],
        out_specs=pl.BlockSpec((2, ROWT, FW), lambda i: (0, i, 0)),
        out_shape=jax.ShapeDtypeStruct((2, npad, FW), jnp.bfloat16),
    )(v, svec)


def _cheb_mix(t, v1, v2, svec, W, b, npad, split_v, emit_next):
    Fin = t.shape[1]
    H = W.shape[2]
    H2 = H // 2
    b2d = b.reshape(1, H)

    def body(t_ref, v1_ref, v2_ref, s_ref, w_ref, b_ref, *out_refs):
        sv = s_ref[...]
        tt = t_ref[...]
        if split_v:
            v1c = jnp.concatenate([v1_ref[0], v1_ref[1]], axis=1)
            v2c = jnp.concatenate([v2_ref[0], v2_ref[1]], axis=1)
        else:
            v1c = v1_ref[0]
            v2c = v2_ref[0]
        v1c = v1c.astype(jnp.float32)
        v2c = v2c.astype(jnp.float32)
        w = w_ref[...].astype(jnp.bfloat16)
        tb = tt.astype(jnp.bfloat16)
        acc = jnp.dot(tb, w[0], precision=_HI, preferred_element_type=jnp.float32)
        acc = acc - jnp.dot((sv * v1c).astype(jnp.bfloat16), w[1], precision=_HI,
                            preferred_element_type=jnp.float32)
        acc = acc + jnp.dot((2.0 * (sv * v2c) - tt).astype(jnp.bfloat16), w[2],
                            precision=_HI, preferred_element_type=jnp.float32)
        h = jnp.maximum(acc + b_ref[...], 0.0)
        out_refs[0][...] = h
        if emit_next:
            u = (sv * h).astype(jnp.bfloat16)
            out_refs[1][0] = u[:, :H2]
            out_refs[1][1] = u[:, H2:]

    nv = 2 if split_v else 1
    in_specs = [
        pl.BlockSpec((ROWT, Fin), lambda i: (i, 0)),
        pl.BlockSpec((nv, ROWT, FW), lambda i: (0, i, 0)),
        pl.BlockSpec((nv, ROWT, FW), lambda i: (0, i, 0)),
        pl.BlockSpec((ROWT, 1), lambda i: (i, 0)),
        pl.BlockSpec(W.shape, lambda i: (0, 0, 0)),
        pl.BlockSpec((1, H), lambda i: (0, 0)),
    ]
    out_shape = [jax.ShapeDtypeStruct((npad, H), jnp.float32)]
    out_specs = [pl.BlockSpec((ROWT, H), lambda i: (i, 0))]
    if emit_next:
        out_shape.append(jax.ShapeDtypeStruct((2, npad, H2), jnp.bfloat16))
        out_specs.append(pl.BlockSpec((2, ROWT, H2), lambda i: (0, i, 0)))

    res = pl.pallas_call(
        body,
        grid=(npad // ROWT,),
        in_specs=in_specs,
        out_specs=out_specs,
        out_shape=out_shape,
    )(t, v1, v2, svec, W, b2d)
    return res if emit_next else res[0]


def kernel(x, edge_index, W1, b1, W2, b2):
    N, IN = x.shape
    H = W1.shape[2]
    E = edge_index.shape[1]

    npad = ((N + 2048) // 2048) * 2048
    nblk = -(-E // (NSUB * EBLK))
    ep = NSUB * nblk * EBLK

    row = edge_index[0]
    col = edge_index[1]
    pad = ep - E
    zi = jnp.zeros((pad,), jnp.int32)
    ti = jnp.full((pad,), npad - 1, jnp.int32)
    src_p = jnp.concatenate([row, zi]).reshape(NSUB, nblk, EBLK)
    dst_p = jnp.concatenate([col, ti]).reshape(NSUB, nblk, EBLK)

    xp = jnp.zeros((npad, IN), jnp.float32).at[:N].set(x)

    zbuf = jnp.zeros((EBLK, FW), jnp.bfloat16)
    aprop = _make_aprop(nblk, npad)

    ep32 = 512 * (-(-E // 512))
    rows32 = jnp.concatenate(
        [row, jnp.full((ep32 - E,), 128 * 128 - 1, jnp.int32)]).reshape(32, -1)
    ididx = jnp.arange(128, dtype=jnp.int32).reshape(1, 128)
    zf = jnp.zeros((128, 128), jnp.float32)
    degq = _make_deghist(ep32 // 32)(rows32, ididx, zf)
    deg = (degq[0] + degq[1]).reshape(-1)[:npad]

    s = jnp.where(deg > 0, jax.lax.rsqrt(jnp.where(deg > 0, deg, 1.0)), 0.0)
    sc = s.reshape(npad, 1)
    s2c = (s * s).reshape(npad, 1)

    u0 = _rowscale_dup(xp, sc, npad)
    v1 = aprop(u0, src_p, dst_p, zbuf)
    u1 = _rowscale_dup0(v1, s2c, npad)
    v2 = aprop(u1, src_p, dst_p, zbuf)
    h, u0b = _cheb_mix(xp, v1, v2, sc, W1, b1, npad, False, True)

    v1b = aprop(u0b, src_p, dst_p, zbuf)
    u1b = _rowscale_stacked(v1b, s2c, npad)
    v2b = aprop(u1b, src_p, dst_p, zbuf)
    out = _cheb_mix(h, v1b, v2b, sc, W2, b2, npad, True, False)

    return out[:N]
